# baseline (speedup 1.0000x reference)
import jax
import jax.numpy as jnp
from jax import lax
from jax.experimental import pallas as pl
from jax.experimental.pallas import tpu as pltpu
from jax.experimental.pallas import tpu_sc as plsc

N = 10000
E = 320000
H = 128
NC = 2
NS = 16
NW = NC * NS
CH = 128
NCHUNK = 79
BLK = 8
NBLK = 10
TOTBLK = 320
EPW = NCHUNK * CH
EP = NW * EPW
NPAD = 10240
RPT = NPAD // NS
WB = 128
NWB = RPT // WB
CW = 16

F32 = jnp.float32


def _make_agg(q0):
    q1 = 2 * NCHUNK - q0
    assert q0 % 2 == 0 and q1 % 2 == 0 and q0 % 8 and q1 % 8

    def _agg_body(h_hbm, row_hbm, col_hbm, out_hbm, acc, row0, col0, row1,
                  col1, msg0, msg1, gsem0, gsem1, ssem0, ssem1):
        c = lax.axis_index("c")
        s = lax.axis_index("s")

        z16 = jnp.zeros((16,), F32)

        def zrow(i, carry):
            for j in range(H // 16):
                msg0[i, pl.ds(j * 16, 16)] = z16
            return carry

        lax.fori_loop(0, WB, zrow, 0)
        for k in range(NWB):
            pltpu.sync_copy(msg0, acc.at[pl.ds(s * RPT + k * WB, WB)])

        plsc.subcore_barrier()

        wid = s * NC + c
        npair = jnp.where(c == 0, q0 // 2, q1 // 2)
        start_e = jnp.where(c == 0, s * q0, NS * q0 + s * q1) * CH

        def step(i, carry):
            off0 = pl.multiple_of(start_e + (2 * i) * CH, 8)
            off1 = pl.multiple_of(start_e + (2 * i + 1) * CH, 8)
            pltpu.sync_copy(row_hbm.at[pl.ds(off0, CH)], row0)
            pltpu.sync_copy(col_hbm.at[pl.ds(off0, CH)], col0)
            g0 = pltpu.async_copy(h_hbm.at[row0], msg0, gsem0)
            pltpu.sync_copy(row_hbm.at[pl.ds(off1, CH)], row1)
            pltpu.sync_copy(col_hbm.at[pl.ds(off1, CH)], col1)
            g1 = pltpu.async_copy(h_hbm.at[row1], msg1, gsem1)
            g0.wait()
            s0 = pltpu.async_copy(msg0, acc.at[col0], ssem0, add=True)
            g1.wait()
            s1 = pltpu.async_copy(msg1, acc.at[col1], ssem1, add=True)
            s0.wait()
            s1.wait()
            return carry

        lax.fori_loop(0, npair, step, 0)

        plsc.subcore_barrier()

        for k in range(NWB):
            st = s * RPT + k * WB
            pltpu.sync_copy(acc.at[pl.ds(st, WB)], msg0)
            pltpu.sync_copy(msg0, out_hbm.at[c, pl.ds(st, WB)])

    return pl.kernel(
        _agg_body,
        out_type=(jax.ShapeDtypeStruct((NC, NPAD, H), F32),),
        mesh=plsc.VectorSubcoreMesh(core_axis_name="c", subcore_axis_name="s"),
        scratch_types=(
            pltpu.VMEM_SHARED((NPAD, H), F32),
            pltpu.VMEM((CH,), jnp.int32),
            pltpu.VMEM((CH,), jnp.int32),
            pltpu.VMEM((CH,), jnp.int32),
            pltpu.VMEM((CH,), jnp.int32),
            pltpu.VMEM((CH, H), F32),
            pltpu.VMEM((CH, H), F32),
            pltpu.SemaphoreType.DMA,
            pltpu.SemaphoreType.DMA,
            pltpu.SemaphoreType.DMA,
            pltpu.SemaphoreType.DMA,
        ),
    )


_sc_agg = _make_agg(122)


def _cnt_body(col_hbm, out_hbm, cacc, col_blk, msg):
    c = lax.axis_index("c")
    s = lax.axis_index("s")
    wid = s * NC + c

    z16 = jnp.zeros((16,), F32)
    one16 = jnp.full((16,), 1.0, F32)

    def zrow(i, carry):
        for j in range(H // 16):
            msg[i, pl.ds(j * 16, 16)] = z16
        return carry

    lax.fori_loop(0, WB, zrow, 0)
    for k in range(NWB):
        pltpu.sync_copy(msg, cacc.at[pl.ds(s * RPT + k * WB, WB)])
    plsc.subcore_barrier()

    def orow(i, carry):
        for j in range(H // 16):
            msg[i, pl.ds(j * 16, 16)] = one16
        return carry

    lax.fori_loop(0, CH, orow, 0)

    base = wid * EPW

    def step(i, carry):
        off = pl.multiple_of(base + i * CH, 8)
        pltpu.sync_copy(col_hbm.at[pl.ds(off, CH)], col_blk)
        pltpu.sync_copy(msg, cacc.at[col_blk], add=True)
        return carry

    lax.fori_loop(0, NCHUNK, step, 0)
    plsc.subcore_barrier()

    for k in range(NWB):
        start = s * RPT + k * WB
        pltpu.sync_copy(cacc.at[pl.ds(start, WB)], msg)
        pltpu.sync_copy(msg, out_hbm.at[c, pl.ds(start, WB)])


_sc_cnt = pl.kernel(
    _cnt_body,
    out_type=(jax.ShapeDtypeStruct((NC, NPAD, H), F32),),
    mesh=plsc.VectorSubcoreMesh(core_axis_name="c", subcore_axis_name="s"),
    scratch_types=(
        pltpu.VMEM_SHARED((NPAD, H), F32),
        pltpu.VMEM((CH,), jnp.int32),
        pltpu.VMEM((CH, H), F32),
    ),
)


def _transform_body(x_ref, w_ref, b_ref, o_ref):
    o_ref[:] = lax.dot_general(
        x_ref[:], w_ref[:], (((1,), (1,)), ((), ())),
        preferred_element_type=F32) + b_ref[:]


_transform = pl.pallas_call(
    _transform_body,
    out_shape=jax.ShapeDtypeStruct((N, H), F32),
)


def _cnt_compact_body(cp_ref, o_ref):
    o_ref[:] = cp_ref[0, :N, :8] + cp_ref[1, :N, :8]


_cnt_compact = pl.pallas_call(
    _cnt_compact_body,
    out_shape=jax.ShapeDtypeStruct((N, 8), F32),
)


def _agg_h(p_ref, cnt_ref, ht_ref):
    ssum = p_ref[0, :N] + p_ref[1, :N]
    cnt = cnt_ref[:, 0:1]
    mean = ssum / jnp.maximum(cnt, 1.0)
    h = jnp.where(cnt > 0.0, mean, ht_ref[:])
    return jnp.maximum(h, 0.0)


def _post_body(p_ref, cnt_ref, ht_ref, w_ref, b_ref, o_ref):
    h = _agg_h(p_ref, cnt_ref, ht_ref)
    o_ref[:] = lax.dot_general(
        h, w_ref[:], (((1,), (1,)), ((), ())),
        preferred_element_type=F32) + b_ref[:]


_post = pl.pallas_call(
    _post_body,
    out_shape=jax.ShapeDtypeStruct((N, H), F32),
)


def _head_body(p_ref, cnt_ref, ht_ref, wc1_ref, bc1_ref, wc2_ref, bc2_ref,
               o_ref):
    h = _agg_h(p_ref, cnt_ref, ht_ref)
    ones = jnp.ones((1, N), F32)
    g = lax.dot_general(ones, h, (((1,), (0,)), ((), ())),
                        preferred_element_type=F32) * (1.0 / N)
    z = lax.dot_general(g, wc1_ref[:], (((1,), (1,)), ((), ())),
                        preferred_element_type=F32) + bc1_ref[:]
    z = jnp.maximum(z, 0.0)
    z = lax.dot_general(z, wc2_ref[:], (((1,), (1,)), ((), ())),
                        preferred_element_type=F32) + bc2_ref[:]
    m = jnp.max(z, axis=1, keepdims=True)
    e = jnp.exp(z - m)
    o_ref[:] = z - m - jnp.log(jnp.sum(e, axis=1, keepdims=True))


_head = pl.pallas_call(
    _head_body,
    out_shape=jax.ShapeDtypeStruct((1, 10), F32),
)


def kernel(x, edge_index, W1, b1, W2, b2, W3, b3, Wc1, bc1, Wc2, bc2):
    pad = EP - E
    row = jnp.concatenate([edge_index[0], jnp.zeros((pad,), jnp.int32)])
    col = jnp.concatenate([edge_index[1], jnp.full((pad,), N, jnp.int32)])
    b1r = b1.reshape(1, H)
    b2r = b2.reshape(1, H)
    b3r = b3.reshape(1, H)
    bc1r = bc1.reshape(1, H // 2)
    bc2r = bc2.reshape(1, 10)

    (cntp,) = _sc_cnt(col)
    cnt = _cnt_compact(cntp)
    h1t = _transform(x, W1, b1r)
    (p1,) = _sc_agg(h1t, row, col)
    h2t = _post(p1, cnt, h1t, W2, b2r)
    (p2,) = _sc_agg(h2t, row, col)
    h3t = _post(p2, cnt, h2t, W3, b3r)
    (p3,) = _sc_agg(h3t, row, col)
    return _head(p3, cnt, h3t, Wc1, bc1r, Wc2, bc2r)


# speedup vs baseline: 1.0462x; 1.0462x over previous
import jax
import jax.numpy as jnp
from jax import lax
from jax.experimental import pallas as pl
from jax.experimental.pallas import tpu as pltpu
from jax.experimental.pallas import tpu_sc as plsc

N = 10000
E = 320000
H = 128
NC = 2
NS = 16
NW = NC * NS
CH = 128
NCHUNK = 79
BLK = 8
NBLK = 10
TOTBLK = 320
EPW = NCHUNK * CH
EP = NW * EPW
NPAD = 10240
RPT = NPAD // NS
WB = 128
NWB = RPT // WB
CW = 16

F32 = jnp.float32


def _make_agg(q0):
    q1 = 2 * NCHUNK - q0
    assert q0 % 2 == 0 and q1 % 2 == 0 and q0 % 8 and q1 % 8

    def _agg_body(h_hbm, row_hbm, col_hbm, out_hbm, acc, row0, col0, row1,
                  col1, msg0, msg1, gsem0, gsem1, ssem0, ssem1):
        c = lax.axis_index("c")
        s = lax.axis_index("s")

        z16 = jnp.zeros((16,), F32)

        def zrow(i, carry):
            for j in range(H // 16):
                msg0[i, pl.ds(j * 16, 16)] = z16
            return carry

        lax.fori_loop(0, WB, zrow, 0)
        for k in range(NWB):
            pltpu.sync_copy(msg0, acc.at[pl.ds(s * RPT + k * WB, WB)])

        plsc.subcore_barrier()

        wid = s * NC + c
        npair = jnp.where(c == 0, q0 // 2, q1 // 2)
        start_e = jnp.where(c == 0, s * q0, NS * q0 + s * q1) * CH

        def step(i, carry):
            off0 = pl.multiple_of(start_e + (2 * i) * CH, 8)
            off1 = pl.multiple_of(start_e + (2 * i + 1) * CH, 8)
            pltpu.sync_copy(row_hbm.at[pl.ds(off0, CH)], row0)
            pltpu.sync_copy(col_hbm.at[pl.ds(off0, CH)], col0)
            g0 = pltpu.async_copy(h_hbm.at[row0], msg0, gsem0)
            pltpu.sync_copy(row_hbm.at[pl.ds(off1, CH)], row1)
            pltpu.sync_copy(col_hbm.at[pl.ds(off1, CH)], col1)
            g1 = pltpu.async_copy(h_hbm.at[row1], msg1, gsem1)
            g0.wait()
            s0 = pltpu.async_copy(msg0, acc.at[col0], ssem0, add=True)
            g1.wait()
            s1 = pltpu.async_copy(msg1, acc.at[col1], ssem1, add=True)
            s0.wait()
            s1.wait()
            return carry

        lax.fori_loop(0, npair, step, 0)

        plsc.subcore_barrier()

        for k in range(NWB):
            st = s * RPT + k * WB
            pltpu.sync_copy(acc.at[pl.ds(st, WB)], msg0)
            pltpu.sync_copy(msg0, out_hbm.at[c, pl.ds(st, WB)])

    return pl.kernel(
        _agg_body,
        out_type=(jax.ShapeDtypeStruct((NC, NPAD, H), F32),),
        mesh=plsc.VectorSubcoreMesh(core_axis_name="c", subcore_axis_name="s"),
        scratch_types=(
            pltpu.VMEM_SHARED((NPAD, H), F32),
            pltpu.VMEM((CH,), jnp.int32),
            pltpu.VMEM((CH,), jnp.int32),
            pltpu.VMEM((CH,), jnp.int32),
            pltpu.VMEM((CH,), jnp.int32),
            pltpu.VMEM((CH, H), F32),
            pltpu.VMEM((CH, H), F32),
            pltpu.SemaphoreType.DMA,
            pltpu.SemaphoreType.DMA,
            pltpu.SemaphoreType.DMA,
            pltpu.SemaphoreType.DMA,
        ),
    )


_sc_agg = _make_agg(114)


def _cnt_body(col_hbm, out_hbm, cacc, col_blk, msg):
    c = lax.axis_index("c")
    s = lax.axis_index("s")
    wid = s * NC + c

    z16 = jnp.zeros((16,), F32)
    one16 = jnp.full((16,), 1.0, F32)

    def zrow(i, carry):
        for j in range(H // 16):
            msg[i, pl.ds(j * 16, 16)] = z16
        return carry

    lax.fori_loop(0, WB, zrow, 0)
    for k in range(NWB):
        pltpu.sync_copy(msg, cacc.at[pl.ds(s * RPT + k * WB, WB)])
    plsc.subcore_barrier()

    def orow(i, carry):
        for j in range(H // 16):
            msg[i, pl.ds(j * 16, 16)] = one16
        return carry

    lax.fori_loop(0, CH, orow, 0)

    base = wid * EPW

    def step(i, carry):
        off = pl.multiple_of(base + i * CH, 8)
        pltpu.sync_copy(col_hbm.at[pl.ds(off, CH)], col_blk)
        pltpu.sync_copy(msg, cacc.at[col_blk], add=True)
        return carry

    lax.fori_loop(0, NCHUNK, step, 0)
    plsc.subcore_barrier()

    for k in range(NWB):
        start = s * RPT + k * WB
        pltpu.sync_copy(cacc.at[pl.ds(start, WB)], msg)
        pltpu.sync_copy(msg, out_hbm.at[c, pl.ds(start, WB)])


_sc_cnt = pl.kernel(
    _cnt_body,
    out_type=(jax.ShapeDtypeStruct((NC, NPAD, H), F32),),
    mesh=plsc.VectorSubcoreMesh(core_axis_name="c", subcore_axis_name="s"),
    scratch_types=(
        pltpu.VMEM_SHARED((NPAD, H), F32),
        pltpu.VMEM((CH,), jnp.int32),
        pltpu.VMEM((CH, H), F32),
    ),
)


def _transform_body(x_ref, w_ref, b_ref, o_ref):
    o_ref[:] = lax.dot_general(
        x_ref[:], w_ref[:], (((1,), (1,)), ((), ())),
        preferred_element_type=F32) + b_ref[:]


_transform = pl.pallas_call(
    _transform_body,
    out_shape=jax.ShapeDtypeStruct((N, H), F32),
)


def _cnt_compact_body(cp_ref, o_ref):
    o_ref[:] = cp_ref[0, :N, :8] + cp_ref[1, :N, :8]


_cnt_compact = pl.pallas_call(
    _cnt_compact_body,
    out_shape=jax.ShapeDtypeStruct((N, 8), F32),
)


def _agg_h(p_ref, cnt_ref, ht_ref):
    ssum = p_ref[0, :N] + p_ref[1, :N]
    cnt = cnt_ref[:, 0:1]
    mean = ssum / jnp.maximum(cnt, 1.0)
    h = jnp.where(cnt > 0.0, mean, ht_ref[:])
    return jnp.maximum(h, 0.0)


def _post_body(p_ref, cnt_ref, ht_ref, w_ref, b_ref, o_ref):
    h = _agg_h(p_ref, cnt_ref, ht_ref)
    o_ref[:] = lax.dot_general(
        h, w_ref[:], (((1,), (1,)), ((), ())),
        preferred_element_type=F32) + b_ref[:]


_post = pl.pallas_call(
    _post_body,
    out_shape=jax.ShapeDtypeStruct((N, H), F32),
)


def _head_body(p_ref, cnt_ref, ht_ref, wc1_ref, bc1_ref, wc2_ref, bc2_ref,
               o_ref):
    h = _agg_h(p_ref, cnt_ref, ht_ref)
    ones = jnp.ones((1, N), F32)
    g = lax.dot_general(ones, h, (((1,), (0,)), ((), ())),
                        preferred_element_type=F32) * (1.0 / N)
    z = lax.dot_general(g, wc1_ref[:], (((1,), (1,)), ((), ())),
                        preferred_element_type=F32) + bc1_ref[:]
    z = jnp.maximum(z, 0.0)
    z = lax.dot_general(z, wc2_ref[:], (((1,), (1,)), ((), ())),
                        preferred_element_type=F32) + bc2_ref[:]
    m = jnp.max(z, axis=1, keepdims=True)
    e = jnp.exp(z - m)
    o_ref[:] = z - m - jnp.log(jnp.sum(e, axis=1, keepdims=True))


_head = pl.pallas_call(
    _head_body,
    out_shape=jax.ShapeDtypeStruct((1, 10), F32),
)


def kernel(x, edge_index, W1, b1, W2, b2, W3, b3, Wc1, bc1, Wc2, bc2):
    pad = EP - E
    row = jnp.concatenate([edge_index[0], jnp.zeros((pad,), jnp.int32)])
    col = jnp.concatenate([edge_index[1], jnp.full((pad,), N, jnp.int32)])
    b1r = b1.reshape(1, H)
    b2r = b2.reshape(1, H)
    b3r = b3.reshape(1, H)
    bc1r = bc1.reshape(1, H // 2)
    bc2r = bc2.reshape(1, 10)

    (cntp,) = _sc_cnt(col)
    cnt = _cnt_compact(cntp)
    h1t = _transform(x, W1, b1r)
    (p1,) = _sc_agg(h1t, row, col)
    h2t = _post(p1, cnt, h1t, W2, b2r)
    (p2,) = _sc_agg(h2t, row, col)
    h3t = _post(p2, cnt, h2t, W3, b3r)
    (p3,) = _sc_agg(h3t, row, col)
    return _head(p3, cnt, h3t, Wc1, bc1r, Wc2, bc2r)
